# Initial kernel scaffold; baseline (speedup 1.0000x reference)
import jax
import jax.numpy as jnp
from jax.experimental import pallas as pl


def kernel(proposals, objectness):
    raise NotImplementedError("write your pallas kernel here")


# speedup vs baseline: 30.1851x; 30.1851x over previous
import jax
import jax.numpy as jnp
from jax.experimental import pallas as pl
from jax.experimental.pallas import tpu as pltpu

_IMG_H, _IMG_W = 800.0, 800.0
_PRE_NMS = 2000
_POST_NMS = 1000
_NMS_THRESH = 0.7
_N_PAD = 2048
_TILE = 128
_N_TILES = _N_PAD // _TILE


def _iou(ax1, ay1, ax2, ay2, a_area, bx1, by1, bx2, by2, b_area):
    lt_x = jnp.maximum(ax1, bx1)
    lt_y = jnp.maximum(ay1, by1)
    rb_x = jnp.minimum(ax2, bx2)
    rb_y = jnp.minimum(ay2, by2)
    w = jnp.clip(rb_x - lt_x, 0.0)
    h = jnp.clip(rb_y - lt_y, 0.0)
    inter = w * h
    return inter / (a_area + b_area - inter + 1e-9)


def _nms_tile_kernel(cx1, cy1, cx2, cy2,
                     rx1, ry1, rx2, ry2,
                     tx1, ty1, tx2, ty2,
                     keep_out,
                     keep_ref):
    t = pl.program_id(1)

    @pl.when(t == 0)
    def _init():
        keep_ref[...] = jnp.zeros_like(keep_ref)

    cols = [cx1[0], cy1[0], cx2[0], cy2[0]]
    rows = [rx1[0], ry1[0], rx2[0], ry2[0]]
    tile = [tx1[0, 0], ty1[0, 0], tx2[0, 0], ty2[0, 0]]
    area_c = (cols[2] - cols[0]) * (cols[3] - cols[1])
    area_r = (rows[2] - rows[0]) * (rows[3] - rows[1])
    area_t = (tile[2] - tile[0]) * (tile[3] - tile[1])

    iou_cross = _iou(rows[0], rows[1], rows[2], rows[3], area_r,
                     cols[0], cols[1], cols[2], cols[3], area_c)
    a_cross = jnp.where(iou_cross > _NMS_THRESH, 1.0, 0.0)
    sup_prev = jax.lax.dot_general(
        a_cross, keep_ref[...],
        dimension_numbers=(((1,), (0,)), ((), ())),
        preferred_element_type=jnp.float32)
    base = jnp.where(sup_prev > 0.5, 0.0, 1.0)

    iou_local = _iou(rows[0], rows[1], rows[2], rows[3], area_r,
                     tile[0], tile[1], tile[2], tile[3], area_t)
    row_ids = jax.lax.broadcasted_iota(jnp.int32, (_TILE, _TILE), 0)
    col_ids = jax.lax.broadcasted_iota(jnp.int32, (_TILE, _TILE), 1)
    s_local = jnp.where((iou_local > _NMS_THRESH) & (col_ids < row_ids),
                        1.0, 0.0)

    def cond(st):
        _, changed, it = st
        return changed & (it < _TILE)

    def body(st):
        k, _, it = st
        sup = jax.lax.dot_general(
            s_local, k,
            dimension_numbers=(((1,), (0,)), ((), ())),
            preferred_element_type=jnp.float32)
        newk = jnp.where(sup > 0.5, 0.0, base)
        changed = jnp.any(newk != k)
        return newk, changed, it + 1

    k0 = base
    k, _, _ = jax.lax.while_loop(cond, body, (k0, jnp.bool_(True), 0))

    keep_ref[pl.ds(t * _TILE, _TILE), :] = k
    keep_out[...] = k.reshape(1, _TILE, 1)


def _run_nms(x1, y1, x2, y2):
    b = x1.shape[0]
    cols_spec = pl.BlockSpec((1, 1, _N_PAD), lambda i, t: (i, 0, 0))
    rows_spec = pl.BlockSpec((1, _TILE, 1), lambda i, t: (i, t, 0))
    tile_spec = pl.BlockSpec((1, 1, 1, _TILE), lambda i, t: (i, t, 0, 0))

    cols = [c[:, None, :] for c in (x1, y1, x2, y2)]
    rows = [c[:, :, None] for c in (x1, y1, x2, y2)]
    tile = [c.reshape(b, _N_TILES, 1, _TILE) for c in (x1, y1, x2, y2)]

    keep = pl.pallas_call(
        _nms_tile_kernel,
        grid=(b, _N_TILES),
        in_specs=[cols_spec] * 4 + [rows_spec] * 4 + [tile_spec] * 4,
        out_specs=pl.BlockSpec((1, _TILE, 1), lambda i, t: (i, t, 0)),
        out_shape=jax.ShapeDtypeStruct((b, _N_PAD, 1), jnp.float32),
        scratch_shapes=[pltpu.VMEM((_N_PAD, 1), jnp.float32)],
        compiler_params=pltpu.CompilerParams(
            dimension_semantics=("arbitrary", "arbitrary")),
    )(*cols, *rows, *tile)
    return keep[:, :, 0]


def kernel(proposals, objectness):
    obj_top, top_idx = jax.lax.top_k(objectness, _PRE_NMS)
    boxes_top = jnp.take_along_axis(proposals, top_idx[..., None], axis=1)

    x1 = jnp.clip(boxes_top[..., 0], 0.0, _IMG_W)
    y1 = jnp.clip(boxes_top[..., 1], 0.0, _IMG_H)
    x2 = jnp.clip(boxes_top[..., 2], 0.0, _IMG_W)
    y2 = jnp.clip(boxes_top[..., 3], 0.0, _IMG_H)
    probs = jax.nn.sigmoid(obj_top)

    pad = _N_PAD - _PRE_NMS
    xp, yp, x2p, y2p = (jnp.pad(c, ((0, 0), (0, pad))) for c in (x1, y1, x2, y2))

    keep = _run_nms(xp, yp, x2p, y2p)[:, :_PRE_NMS]

    kept_scores = jnp.where(keep > 0.5, probs, -1e9)
    top_scores, sel = jax.lax.top_k(kept_scores, _POST_NMS)
    boxes_clipped = jnp.stack([x1, y1, x2, y2], axis=-1)
    top_boxes = jnp.take_along_axis(boxes_clipped, sel[..., None], axis=1)
    return jnp.concatenate([top_boxes, top_scores[..., None]], axis=-1)


# speedup vs baseline: 32.9156x; 1.0905x over previous
import jax
import jax.numpy as jnp
from jax.experimental import pallas as pl
from jax.experimental.pallas import tpu as pltpu

_IMG_H, _IMG_W = 800.0, 800.0
_PRE_NMS = 2000
_POST_NMS = 1000
_NMS_THRESH = 0.7
_N_PAD = 2048
_TILE = 128
_N_TILES = _N_PAD // _TILE
_BATCH_DIMS = (((2,), (1,)), ((0,), (0,)))


def _iou(ax1, ay1, ax2, ay2, a_area, bx1, by1, bx2, by2, b_area):
    lt_x = jnp.maximum(ax1, bx1)
    lt_y = jnp.maximum(ay1, by1)
    rb_x = jnp.minimum(ax2, bx2)
    rb_y = jnp.minimum(ay2, by2)
    w = jnp.clip(rb_x - lt_x, 0.0)
    h = jnp.clip(rb_y - lt_y, 0.0)
    inter = w * h
    return inter / (a_area + b_area - inter + 1e-9)


def _nms_tile_kernel(cx1, cy1, cx2, cy2,
                     rx1, ry1, rx2, ry2,
                     tx1, ty1, tx2, ty2,
                     keep_out,
                     keep_ref):
    t = pl.program_id(0)
    b = keep_ref.shape[0]

    @pl.when(t == 0)
    def _init():
        keep_ref[...] = jnp.zeros_like(keep_ref)

    cols = [cx1[...], cy1[...], cx2[...], cy2[...]]
    rows = [rx1[...], ry1[...], rx2[...], ry2[...]]
    tile = [v.reshape(b, 1, _TILE) for v in (tx1[...], ty1[...], tx2[...], ty2[...])]
    area_c = (cols[2] - cols[0]) * (cols[3] - cols[1])
    area_r = (rows[2] - rows[0]) * (rows[3] - rows[1])
    area_t = (tile[2] - tile[0]) * (tile[3] - tile[1])

    iou_cross = _iou(rows[0], rows[1], rows[2], rows[3], area_r,
                     cols[0], cols[1], cols[2], cols[3], area_c)
    a_cross = jnp.where(iou_cross > _NMS_THRESH, 1.0, 0.0)
    sup_prev = jax.lax.dot_general(
        a_cross, keep_ref[...],
        dimension_numbers=_BATCH_DIMS,
        preferred_element_type=jnp.float32)
    base = jnp.where(sup_prev > 0.5, 0.0, 1.0)

    iou_local = _iou(rows[0], rows[1], rows[2], rows[3], area_r,
                     tile[0], tile[1], tile[2], tile[3], area_t)
    row_ids = jax.lax.broadcasted_iota(jnp.int32, (1, _TILE, _TILE), 1)
    col_ids = jax.lax.broadcasted_iota(jnp.int32, (1, _TILE, _TILE), 2)
    s_local = jnp.where((iou_local > _NMS_THRESH) & (col_ids < row_ids),
                        1.0, 0.0)

    def cond(st):
        _, changed, it = st
        return changed & (it < _TILE)

    def body(st):
        k, _, it = st
        sup = jax.lax.dot_general(
            s_local, k,
            dimension_numbers=_BATCH_DIMS,
            preferred_element_type=jnp.float32)
        newk = jnp.where(sup > 0.5, 0.0, base)
        changed = jnp.any(newk != k)
        return newk, changed, it + 1

    k0 = base
    k, _, _ = jax.lax.while_loop(cond, body, (k0, jnp.bool_(True), 0))

    keep_ref[:, pl.ds(t * _TILE, _TILE), :] = k
    keep_out[...] = k


def _run_nms(x1, y1, x2, y2):
    b = x1.shape[0]
    cols_spec = pl.BlockSpec((b, 1, _N_PAD), lambda t: (0, 0, 0))
    rows_spec = pl.BlockSpec((b, _TILE, 1), lambda t: (0, t, 0))
    tile_spec = pl.BlockSpec((b, 1, 1, _TILE), lambda t: (0, t, 0, 0))

    cols = [c[:, None, :] for c in (x1, y1, x2, y2)]
    rows = [c[:, :, None] for c in (x1, y1, x2, y2)]
    tile = [c.reshape(b, _N_TILES, 1, _TILE) for c in (x1, y1, x2, y2)]

    keep = pl.pallas_call(
        _nms_tile_kernel,
        grid=(_N_TILES,),
        in_specs=[cols_spec] * 4 + [rows_spec] * 4 + [tile_spec] * 4,
        out_specs=pl.BlockSpec((b, _TILE, 1), lambda t: (0, t, 0)),
        out_shape=jax.ShapeDtypeStruct((b, _N_PAD, 1), jnp.float32),
        scratch_shapes=[pltpu.VMEM((b, _N_PAD, 1), jnp.float32)],
        compiler_params=pltpu.CompilerParams(
            dimension_semantics=("arbitrary",)),
    )(*cols, *rows, *tile)
    return keep[:, :, 0]


def kernel(proposals, objectness):
    obj_top, top_idx = jax.lax.top_k(objectness, _PRE_NMS)
    boxes_top = jnp.take_along_axis(proposals, top_idx[..., None], axis=1)

    x1 = jnp.clip(boxes_top[..., 0], 0.0, _IMG_W)
    y1 = jnp.clip(boxes_top[..., 1], 0.0, _IMG_H)
    x2 = jnp.clip(boxes_top[..., 2], 0.0, _IMG_W)
    y2 = jnp.clip(boxes_top[..., 3], 0.0, _IMG_H)
    probs = jax.nn.sigmoid(obj_top)

    pad = _N_PAD - _PRE_NMS
    xp, yp, x2p, y2p = (jnp.pad(c, ((0, 0), (0, pad))) for c in (x1, y1, x2, y2))

    keep = _run_nms(xp, yp, x2p, y2p)[:, :_PRE_NMS]

    kept_scores = jnp.where(keep > 0.5, probs, -1e9)
    top_scores, sel = jax.lax.top_k(kept_scores, _POST_NMS)
    boxes_clipped = jnp.stack([x1, y1, x2, y2], axis=-1)
    top_boxes = jnp.take_along_axis(boxes_clipped, sel[..., None], axis=1)
    return jnp.concatenate([top_boxes, top_scores[..., None]], axis=-1)


# speedup vs baseline: 36.9334x; 1.1221x over previous
import jax
import jax.numpy as jnp
from jax.experimental import pallas as pl
from jax.experimental.pallas import tpu as pltpu

_IMG_H, _IMG_W = 800.0, 800.0
_PRE_NMS = 2000
_POST_NMS = 1000
_NMS_THRESH = 0.7
_N_PAD = 2048
_TILE = 128
_N_TILES = _N_PAD // _TILE
_BATCH_DIMS = (((2,), (1,)), ((0,), (0,)))


def _iou(ax1, ay1, ax2, ay2, a_area, bx1, by1, bx2, by2, b_area):
    lt_x = jnp.maximum(ax1, bx1)
    lt_y = jnp.maximum(ay1, by1)
    rb_x = jnp.minimum(ax2, bx2)
    rb_y = jnp.minimum(ay2, by2)
    w = jnp.clip(rb_x - lt_x, 0.0)
    h = jnp.clip(rb_y - lt_y, 0.0)
    inter = w * h
    return inter / (a_area + b_area - inter + 1e-9)


def _nms_tile_kernel(cx1, cy1, cx2, cy2,
                     rx1, ry1, rx2, ry2,
                     tx1, ty1, tx2, ty2,
                     keep_out,
                     keep_ref):
    t = pl.program_id(0)
    b = keep_ref.shape[0]

    @pl.when(t == 0)
    def _init():
        keep_ref[...] = jnp.zeros_like(keep_ref)

    cols = [cx1[...], cy1[...], cx2[...], cy2[...]]
    rows = [rx1[...], ry1[...], rx2[...], ry2[...]]
    tile = [v.reshape(b, 1, _TILE) for v in (tx1[...], ty1[...], tx2[...], ty2[...])]
    area_c = (cols[2] - cols[0]) * (cols[3] - cols[1])
    area_r = (rows[2] - rows[0]) * (rows[3] - rows[1])
    area_t = (tile[2] - tile[0]) * (tile[3] - tile[1])

    iou_cross = _iou(rows[0], rows[1], rows[2], rows[3], area_r,
                     cols[0], cols[1], cols[2], cols[3], area_c)
    a_cross = jnp.where(iou_cross > _NMS_THRESH, 1.0, 0.0)
    sup_prev = jax.lax.dot_general(
        a_cross, keep_ref[...],
        dimension_numbers=_BATCH_DIMS,
        preferred_element_type=jnp.float32)
    base = jnp.where(sup_prev > 0.5, 0.0, 1.0)

    iou_local = _iou(rows[0], rows[1], rows[2], rows[3], area_r,
                     tile[0], tile[1], tile[2], tile[3], area_t)
    row_ids = jax.lax.broadcasted_iota(jnp.int32, (1, _TILE, _TILE), 1)
    col_ids = jax.lax.broadcasted_iota(jnp.int32, (1, _TILE, _TILE), 2)
    s_local = jnp.where((iou_local > _NMS_THRESH) & (col_ids < row_ids),
                        1.0, 0.0)

    def cond(st):
        _, changed, it = st
        return changed & (it < _TILE)

    def body(st):
        k, _, it = st
        sup = jax.lax.dot_general(
            s_local, k,
            dimension_numbers=_BATCH_DIMS,
            preferred_element_type=jnp.float32)
        newk = jnp.where(sup > 0.5, 0.0, base)
        changed = jnp.any(newk != k)
        return newk, changed, it + 1

    k0 = base
    k, _, _ = jax.lax.while_loop(cond, body, (k0, jnp.bool_(True), 0))

    keep_ref[:, pl.ds(t * _TILE, _TILE), :] = k
    keep_out[...] = k


def _run_nms(x1, y1, x2, y2):
    b = x1.shape[0]
    cols_spec = pl.BlockSpec((b, 1, _N_PAD), lambda t: (0, 0, 0))
    rows_spec = pl.BlockSpec((b, _TILE, 1), lambda t: (0, t, 0))
    tile_spec = pl.BlockSpec((b, 1, 1, _TILE), lambda t: (0, t, 0, 0))

    cols = [c[:, None, :] for c in (x1, y1, x2, y2)]
    rows = [c[:, :, None] for c in (x1, y1, x2, y2)]
    tile = [c.reshape(b, _N_TILES, 1, _TILE) for c in (x1, y1, x2, y2)]

    keep = pl.pallas_call(
        _nms_tile_kernel,
        grid=(_N_TILES,),
        in_specs=[cols_spec] * 4 + [rows_spec] * 4 + [tile_spec] * 4,
        out_specs=pl.BlockSpec((b, _TILE, 1), lambda t: (0, t, 0)),
        out_shape=jax.ShapeDtypeStruct((b, _N_PAD, 1), jnp.float32),
        scratch_shapes=[pltpu.VMEM((b, _N_PAD, 1), jnp.float32)],
        compiler_params=pltpu.CompilerParams(
            dimension_semantics=("arbitrary",)),
    )(*cols, *rows, *tile)
    return keep[:, :, 0]


def kernel(proposals, objectness):
    cand_vals, cand_idx = jax.lax.approx_max_k(objectness, _N_PAD,
                                               recall_target=1.0)
    neg_sorted, idx_sorted = jax.lax.sort(
        (-cand_vals, cand_idx.astype(jnp.int32)), dimension=-1, num_keys=2)
    obj_top = -neg_sorted[:, :_PRE_NMS]
    top_idx = idx_sorted[:, :_PRE_NMS]
    boxes_top = jnp.take_along_axis(proposals, top_idx[..., None], axis=1)

    x1 = jnp.clip(boxes_top[..., 0], 0.0, _IMG_W)
    y1 = jnp.clip(boxes_top[..., 1], 0.0, _IMG_H)
    x2 = jnp.clip(boxes_top[..., 2], 0.0, _IMG_W)
    y2 = jnp.clip(boxes_top[..., 3], 0.0, _IMG_H)
    probs = jax.nn.sigmoid(obj_top)

    pad = _N_PAD - _PRE_NMS
    xp, yp, x2p, y2p = (jnp.pad(c, ((0, 0), (0, pad))) for c in (x1, y1, x2, y2))

    keep = _run_nms(xp, yp, x2p, y2p)[:, :_PRE_NMS]

    kept_scores = jnp.where(keep > 0.5, probs, -1e9)
    top_scores, sel = jax.lax.top_k(kept_scores, _POST_NMS)
    boxes_clipped = jnp.stack([x1, y1, x2, y2], axis=-1)
    top_boxes = jnp.take_along_axis(boxes_clipped, sel[..., None], axis=1)
    return jnp.concatenate([top_boxes, top_scores[..., None]], axis=-1)
